# baseline (speedup 1.0000x reference)
import functools

import jax
import jax.numpy as jnp
from jax import lax
from jax.experimental import pallas as pl
from jax.experimental.pallas import tpu as pltpu
from jax.experimental.pallas import tpu_sc as plsc

VOCAB = 1000000
D = 64
B = 1024
S = 200
N = B * S
EPS = 1e-12

NC, NS = 2, 16
NW = NC * NS
BPW = N // NW
CH = 128
NCH = BPW // CH

_sc_mesh = plsc.VectorSubcoreMesh(core_axis_name="c", subcore_axis_name="s")


@functools.partial(
    pl.kernel,
    out_type=jax.ShapeDtypeStruct((N, D), jnp.float32),
    mesh=_sc_mesh,
    compiler_params=pltpu.CompilerParams(use_tc_tiling_on_sc=False),
    scratch_types=[
        pltpu.VMEM((BPW,), jnp.int32),
        pltpu.VMEM((CH, D), jnp.float32),
        pltpu.VMEM((CH, D), jnp.float32),
        pltpu.SemaphoreType.DMA,
        pltpu.SemaphoreType.DMA,
    ],
)
def _sc_gather(ids_hbm, table_hbm, out_hbm, idx_v, rows0, rows1, sem0, sem1):
    wid = lax.axis_index("s") * NC + lax.axis_index("c")
    base = wid * BPW
    pltpu.sync_copy(ids_hbm.at[pl.ds(base, BPW)], idx_v)

    rows = (rows0, rows1)
    sems = (sem0, sem1)

    first = pltpu.async_copy(table_hbm.at[idx_v.at[pl.ds(0, CH)]], rows0, sem0)

    def body(c, _):
        for b in range(2):
            cb = (c * 2 + b) * CH
            nb = cb + CH

            @pl.when(nb < NCH * CH)
            def _():
                pltpu.async_copy(
                    table_hbm.at[idx_v.at[pl.ds(nb, CH)]],
                    rows[1 - b], sems[1 - b],
                )

            pltpu.make_async_copy(
                table_hbm.at[idx_v.at[pl.ds(cb, CH)]], rows[b], sems[b]
            ).wait()
            pltpu.sync_copy(rows[b], out_hbm.at[pl.ds(base + cb, CH)])

        return 0

    lax.fori_loop(0, NCH // 2, body, 0)
    del first


VB = 32768


def _tp_body(x_ref, eye_ref, o_ref):
    x = x_ref[...]
    y = lax.dot_general(
        x, eye_ref[...], (((0,), (0,)), ((), ())),
        preferred_element_type=jnp.float32,
    )
    o_ref[:, :D] = y[: VB // 2]
    o_ref[:, D:] = y[VB // 2 :]


NTB = (VOCAB + VB - 1) // VB


def _transpose_fold(wt_t, eye):
    return pl.pallas_call(
        _tp_body,
        out_shape=jax.ShapeDtypeStruct((NTB * VB // 2, 2 * D), jnp.float32),
        grid=(NTB,),
        in_specs=[
            pl.BlockSpec((D, VB), lambda i: (0, i)),
            pl.BlockSpec((D, D), lambda i: (0, 0)),
        ],
        out_specs=pl.BlockSpec((VB // 2, 2 * D), lambda i: (i, 0)),
    )(wt_t, eye)


def _ln_body(g_ref, pe_ref, gam_ref, bet_ref, o_ref):
    x = g_ref[...] + pe_ref[...]
    xl, xr = x[:, :D], x[:, D:]
    ml = jnp.mean(xl, axis=-1, keepdims=True)
    vl = jnp.mean(jnp.square(xl - ml), axis=-1, keepdims=True)
    yl = (xl - ml) * lax.rsqrt(vl + EPS)
    mr = jnp.mean(xr, axis=-1, keepdims=True)
    vr = jnp.mean(jnp.square(xr - mr), axis=-1, keepdims=True)
    yr = (xr - mr) * lax.rsqrt(vr + EPS)
    o_ref[...] = jnp.concatenate([yl, yr], axis=1) * gam_ref[0] + bet_ref[0]


RB2 = 3200


def kernel(input_ids, word_table, pe, gamma, beta):
    ids_flat = input_ids.reshape(N).astype(jnp.int32)
    h = VB // 2
    idx64 = (ids_flat // VB) * VB + (ids_flat % h) * 2 + (ids_flat // h) % 2
    table2 = _transpose_fold(word_table.T, jnp.eye(D, dtype=jnp.float32))
    table64 = table2.reshape(NTB * VB, D)

    gathered = _sc_gather(idx64, table64)
    g2 = gathered.reshape(N // 2, 2 * D)

    pe_pairs = jnp.tile(pe.reshape(S // 2, 2 * D), (2 * RB2 // S, 1))
    gam2 = jnp.tile(gamma, 2).reshape(1, 2 * D)
    bet2 = jnp.tile(beta, 2).reshape(1, 2 * D)

    out = pl.pallas_call(
        _ln_body,
        out_shape=jax.ShapeDtypeStruct((N // 2, 2 * D), jnp.float32),
        grid=(N // 2 // RB2,),
        in_specs=[
            pl.BlockSpec((RB2, 2 * D), lambda i: (i, 0)),
            pl.BlockSpec((RB2, 2 * D), lambda i: (0, 0)),
            pl.BlockSpec((1, 2 * D), lambda i: (0, 0)),
            pl.BlockSpec((1, 2 * D), lambda i: (0, 0)),
        ],
        out_specs=pl.BlockSpec((RB2, 2 * D), lambda i: (i, 0)),
    )(g2, pe_pairs, gam2, bet2)

    return out.reshape(B, S, D)
